# baseline (speedup 1.0000x reference)
import functools

import jax
import jax.numpy as jnp
from jax import lax
from jax.experimental import pallas as pl
from jax.experimental.pallas import tpu as pltpu
from jax.experimental.pallas import tpu_sc as plsc

_LANES = 16
_NSUB = 32
_K = 32


def _mm_body(a_ref, w_ref, xl_ref, xr_ref, *, hc, elu):
    a = a_ref[...]
    if elu:
        a = jnp.where(a > 0.0, a, jnp.exp(jnp.minimum(a, 0.0)) - 1.0)
    o = lax.dot(a, w_ref[...], preferred_element_type=jnp.float32)
    xl_ref[...] = o[:, :hc]
    xr_ref[...] = o[:, hc:]


def _matmul(a, w, hc, elu):
    np_, kdim = a.shape
    bm = 128
    grid = np_ // bm
    return pl.pallas_call(
        functools.partial(_mm_body, hc=hc, elu=elu),
        grid=(grid,),
        in_specs=[
            pl.BlockSpec((bm, kdim), lambda i: (i, 0)),
            pl.BlockSpec((kdim, 2 * hc), lambda i: (0, 0)),
        ],
        out_specs=[
            pl.BlockSpec((bm, hc), lambda i: (i, 0)),
            pl.BlockSpec((bm, hc), lambda i: (i, 0)),
        ],
        out_shape=[
            jax.ShapeDtypeStruct((np_, hc), jnp.float32),
            jax.ShapeDtypeStruct((np_, hc), jnp.float32),
        ],
    )(a, w)


def _edge_stage(xl2, xr2, src_s, dst_s, ebounds, att_f, bias, heads, ch, np_):
    hc = heads * ch
    nj = ch // _LANES
    n_ranges = _NSUB // heads
    rng_nodes = np_ // n_ranges
    eb_len = ebounds.shape[0]

    mesh = plsc.VectorSubcoreMesh(core_axis_name="c", subcore_axis_name="s")

    @functools.partial(
        pl.kernel,
        mesh=mesh,
        compiler_params=pltpu.CompilerParams(needs_layout_passes=False),
        out_type=jax.ShapeDtypeStruct((np_, hc), jnp.float32),
        scratch_types=[
            pltpu.VMEM((eb_len,), jnp.int32),
            pltpu.VMEM((hc,), jnp.float32),
            pltpu.VMEM((hc,), jnp.float32),
            pltpu.VMEM((_K,), jnp.int32),
            pltpu.VMEM((_K,), jnp.int32),
            pltpu.VMEM((_K + _LANES,), jnp.int32),
            pltpu.VMEM((_K + _LANES,), jnp.int32),
            pltpu.VMEM((_K,), jnp.int32),
            pltpu.VMEM((_K,), jnp.int32),
            pltpu.VMEM((_K,), jnp.int32),
            pltpu.VMEM((_K,), jnp.int32),
            pltpu.VMEM((_K, ch), jnp.float32),
            pltpu.VMEM((_K, ch), jnp.float32),
            pltpu.VMEM((_K, ch), jnp.float32),
            pltpu.VMEM((_K, ch), jnp.float32),
            pltpu.VMEM((ch,), jnp.float32),
            pltpu.VMEM((2 * _LANES,), jnp.float32),
            pltpu.VMEM((2, ch), jnp.float32),
            pltpu.VMEM((8, ch), jnp.float32),
            pltpu.SemaphoreType.DMA,
            pltpu.SemaphoreType.DMA,
            pltpu.SemaphoreType.DMA,
            pltpu.SemaphoreType.DMA,
            pltpu.SemaphoreType.DMA,
        ],
    )
    def edge_kernel(xl_h, xr_h, src_h, dst_h, eb_h, att_h, b_h, out_h,
                    eb_v, att_v, b_v, sidx0, sidx1, dstb0, dstb1,
                    gl0, gl1, gr0, gr1, rl0, rl1, rr0, rr1,
                    acc_v, st_v, orow_v, pre_v,
                    sl0, sl1, sr0, sr1, sem_out):
        sidx = (sidx0, sidx1)
        dstb = (dstb0, dstb1)
        gl = (gl0, gl1)
        gr = (gr0, gr1)
        rl = (rl0, rl1)
        rr = (rr0, rr1)
        sml = (sl0, sl1)
        smr = (sr0, sr1)

        wid = lax.axis_index("s") * 2 + lax.axis_index("c")
        rid = wid // heads
        h = wid - rid * heads
        hch = h * ch
        node0 = rid * rng_nodes
        pltpu.sync_copy(eb_h, eb_v)
        pltpu.sync_copy(att_h, att_v)
        pltpu.sync_copy(b_h, b_v)
        ebp = eb_v[pl.ds(rid * 8, _LANES)]
        e_lo = ebp[0]
        e_hi = ebp[1]
        hoffv = jnp.full((_LANES,), h * np_, jnp.int32)

        zero16 = jnp.zeros((_LANES,), jnp.float32)
        neg16 = jnp.full((_LANES,), -3e38, jnp.float32)
        for j in range(nj):
            acc_v[pl.ds(j * _LANES, _LANES)] = zero16

        for j in range(nj):
            bv = b_v[pl.ds(hch + j * _LANES, _LANES)]
            for r_ in range(8):
                pre_v[r_, pl.ds(j * _LANES, _LANES)] = bv

        def _pre_blk(t, _):
            pltpu.sync_copy(pre_v,
                            out_h.at[pl.ds(node0 + t * 8, 8), pl.ds(hch, ch)])
            return 0
        lax.fori_loop(0, rng_nodes // 8, _pre_blk, 0)

        lane15 = jnp.full((_LANES, 1), 15, jnp.int32)
        _gd = lax.GatherDimensionNumbers(
            offset_dims=(), collapsed_slice_dims=(0,), start_index_map=(0,))

        def _bcast_last(vec):
            return lax.gather(vec, lane15, _gd, slice_sizes=(1,),
                              mode=lax.GatherScatterMode.PROMISE_IN_BOUNDS)

        g0 = e_lo // _K
        g1 = (e_hi + (_K - 1)) // _K

        def _issue(g, b):
            base_e = g * _K
            pltpu.sync_copy(src_h.at[pl.ds(base_e, _K)], sidx[b])
            pltpu.sync_copy(dst_h.at[pl.ds(base_e, _K)],
                            dstb[b].at[pl.ds(0, _K)])
            for q in range(_K // _LANES):
                sl_ = pl.ds(q * _LANES, _LANES)
                gl[b][sl_] = sidx[b][sl_] + hoffv
                gr[b][sl_] = dstb[b][sl_] + hoffv
            pltpu.async_copy(xl_h.at[gl[b]], rl[b], sml[b])
            pltpu.async_copy(xr_h.at[gr[b]], rr[b], smr[b])

        def _chunk(g, b, cf):
            @pl.when(g + 1 < g1)
            def _():
                _issue(g + 1, 1 - b)

            pltpu.make_async_copy(xl_h.at[gl[b]], rl[b], sml[b]).wait()
            pltpu.make_async_copy(xr_h.at[gr[b]], rr[b], smr[b]).wait()
            base_e = g * _K
            dv = dstb[b]
            rlv = rl[b]
            rrv = rr[b]

            def edge_body(i, carry):
                cur = carry[0]
                fcnt = carry[1]
                m = carry[2]
                den = carry[3]
                accs = carry[4:]
                d = dv[pl.ds(i, _LANES)][0]
                is_new = d != cur

                def flush(cf0):
                    cur0, f0 = cf0

                    @pl.when(cur0 >= 0)
                    def _():
                        slot = lax.rem(f0, 2)

                        @pl.when(f0 >= 2)
                        def _():
                            pltpu.make_async_copy(
                                orow_v.at[slot],
                                out_h.at[cur0, pl.ds(hch, ch)],
                                sem_out).wait()

                        inv = 1.0 / (den + 1e-16)
                        for j in range(nj):
                            orow_v[slot, pl.ds(j * _LANES, _LANES)] = (
                                accs[j] * inv
                                + b_v[pl.ds(hch + j * _LANES, _LANES)])
                        pltpu.async_copy(orow_v.at[slot],
                                         out_h.at[cur0, pl.ds(hch, ch)],
                                         sem_out)
                    return (d, jnp.where(cur0 >= 0, f0 + 1, f0))

                cur, fcnt = lax.cond(is_new, flush, lambda c2: c2,
                                     (cur, fcnt))

                m_b = jnp.where(is_new, neg16, m)
                den_b = jnp.where(is_new, zero16, den)
                parts = [zero16] * 8
                ls = []
                for j in range(nj):
                    sl_ = pl.ds(j * _LANES, _LANES)
                    hsl = pl.ds(hch + j * _LANES, _LANES)
                    lj = rlv[i, sl_]
                    ls.append(lj)
                    z = lj + rrv[i, sl_]
                    z = jnp.maximum(z, 0.2 * z)
                    parts[j % 8] = parts[j % 8] + att_v[hsl] * z
                part = parts[0]
                if nj > 8:
                    part = (((parts[0] + parts[1]) + (parts[2] + parts[3]))
                            + ((parts[4] + parts[5]) + (parts[6] + parts[7])))
                lvec = _bcast_last(jnp.cumsum(part))
                mn = jnp.maximum(m_b, lvec)
                r = jnp.exp(m_b - mn)
                w = jnp.exp(lvec - mn)
                den_n = den_b * r + w
                r_eff = jnp.where(is_new, zero16, r)
                new_accs = [accs[j] * r_eff + w * ls[j] for j in range(nj)]
                return (cur, fcnt, mn, den_n, *new_accs)

            ilo = jnp.maximum(e_lo - base_e, 0)
            ihi = jnp.minimum(e_hi - base_e, _K)
            m0 = st_v[pl.ds(0, _LANES)]
            d0 = st_v[pl.ds(_LANES, _LANES)]
            accs0 = [acc_v[pl.ds(j * _LANES, _LANES)] for j in range(nj)]
            res = lax.fori_loop(ilo, ihi, edge_body,
                                (cf[0], cf[1], m0, d0, *accs0))
            st_v[pl.ds(0, _LANES)] = res[2]
            st_v[pl.ds(_LANES, _LANES)] = res[3]
            for j in range(nj):
                acc_v[pl.ds(j * _LANES, _LANES)] = res[4 + j]
            return (res[0], res[1])

        @pl.when(g1 > g0)
        def _():
            _issue(g0, 0)

        def pair_body(t, cf):
            for b in (0, 1):
                g = g0 + 2 * t + b
                cf = lax.cond(g < g1,
                              lambda c, g=g, b=b: _chunk(g, b, c),
                              lambda c: c, cf)
            return cf

        npairs = (g1 - g0 + 1) // 2
        cur, fcnt = lax.fori_loop(0, npairs, pair_body,
                                  (jnp.int32(-1), jnp.int32(0)))

        @pl.when(cur >= 0)
        def _():
            slot = lax.rem(fcnt, 2)

            @pl.when(fcnt >= 2)
            def _():
                pltpu.make_async_copy(orow_v.at[slot],
                                      out_h.at[cur, pl.ds(hch, ch)],
                                      sem_out).wait()

            den = st_v[pl.ds(_LANES, _LANES)]
            inv = 1.0 / (den + 1e-16)
            for j in range(nj):
                orow_v[slot, pl.ds(j * _LANES, _LANES)] = (
                    acc_v[pl.ds(j * _LANES, _LANES)] * inv
                    + b_v[pl.ds(hch + j * _LANES, _LANES)])
            pltpu.async_copy(orow_v.at[slot],
                             out_h.at[cur, pl.ds(hch, ch)], sem_out)

        fin = fcnt + jnp.where(cur >= 0, 1, 0)

        @pl.when(fin >= 1)
        def _():
            pltpu.make_async_copy(orow_v.at[0],
                                  out_h.at[0, pl.ds(hch, ch)],
                                  sem_out).wait()

        @pl.when(fin >= 2)
        def _():
            pltpu.make_async_copy(orow_v.at[0],
                                  out_h.at[0, pl.ds(hch, ch)],
                                  sem_out).wait()

    return edge_kernel(xl2, xr2, src_s, dst_s, ebounds, att_f, bias)


def _head_major(x, heads, ch, np_):
    if heads == 1:
        return x
    return x.reshape(np_, heads, ch).transpose(1, 0, 2).reshape(
        heads * np_, ch)


def kernel(x, edge_index, W1l, W1r, att1, b1, W2l, W2r, att2, b2,
           W3l, W3r, att3, b3):
    n = x.shape[0]
    e = edge_index.shape[1]

    npw = ((n + _NSUB - 1) // _NSUB + 7) // 8 * 8
    np_ = ((npw * _NSUB + 127) // 128) * 128
    npw = np_ // _NSUB

    src = edge_index[0].astype(jnp.int32)
    dst = edge_index[1].astype(jnp.int32)
    order = jnp.argsort(dst)
    src_s = jnp.take(src, order)
    dst_s = jnp.take(dst, order)
    ep = (e + _K - 1) // _K * _K
    if ep != e:
        src_s = jnp.pad(src_s, (0, ep - e))
        dst_s = jnp.pad(dst_s, (0, ep - e), constant_values=n)
    bounds = jnp.arange(_NSUB + 1, dtype=jnp.int32) * npw
    estarts = jnp.searchsorted(dst_s[:e], bounds, side="left").astype(jnp.int32)

    def _ebounds(heads):
        nr = _NSUB // heads
        idx = jnp.arange(nr, dtype=jnp.int32)
        eb = jnp.zeros((nr * 8 + 16,), jnp.int32)
        eb = eb.at[idx * 8].set(estarts[idx * heads])
        eb = eb.at[idx * 8 + 1].set(estarts[(idx + 1) * heads])
        return eb

    xp = jnp.pad(x, ((0, np_ - n), (0, 0)))

    w1 = jnp.concatenate([W1l, W1r], axis=1)
    w2 = jnp.concatenate([W2l, W2r], axis=1)
    w3 = jnp.concatenate([W3l, W3r], axis=1)

    outs = []
    a = xp
    for li, (w, att, b) in enumerate(
            ((w1, att1, b1), (w2, att2, b2), (w3, att3, b3))):
        heads, ch = att.shape
        xl, xr = _matmul(a, w, hc=heads * ch, elu=(li > 0))
        xl2 = _head_major(xl, heads, ch, np_)
        xr2 = _head_major(xr, heads, ch, np_)
        a = _edge_stage(xl2, xr2, src_s, dst_s, _ebounds(heads),
                        att.reshape(-1), b, heads, ch, np_)
    return a[:n]


# speedup vs baseline: 1.1553x; 1.1553x over previous
import functools

import jax
import jax.numpy as jnp
from jax import lax
from jax.experimental import pallas as pl
from jax.experimental.pallas import tpu as pltpu
from jax.experimental.pallas import tpu_sc as plsc

_LANES = 16
_NSUB = 32
_K = 64


def _mm_body(a_ref, w_ref, xl_ref, xr_ref, *, hc, elu):
    a = a_ref[...]
    if elu:
        a = jnp.where(a > 0.0, a, jnp.exp(jnp.minimum(a, 0.0)) - 1.0)
    o = lax.dot(a, w_ref[...], preferred_element_type=jnp.float32)
    xl_ref[...] = o[:, :hc]
    xr_ref[...] = o[:, hc:]


def _matmul(a, w, hc, elu):
    np_, kdim = a.shape
    bm = 128
    grid = np_ // bm
    return pl.pallas_call(
        functools.partial(_mm_body, hc=hc, elu=elu),
        grid=(grid,),
        in_specs=[
            pl.BlockSpec((bm, kdim), lambda i: (i, 0)),
            pl.BlockSpec((kdim, 2 * hc), lambda i: (0, 0)),
        ],
        out_specs=[
            pl.BlockSpec((bm, hc), lambda i: (i, 0)),
            pl.BlockSpec((bm, hc), lambda i: (i, 0)),
        ],
        out_shape=[
            jax.ShapeDtypeStruct((np_, hc), jnp.float32),
            jax.ShapeDtypeStruct((np_, hc), jnp.float32),
        ],
    )(a, w)


def _edge_stage(xl2, xr2, src_s, dst_s, ebounds, att_f, bias, heads, ch, np_):
    hc = heads * ch
    nj = ch // _LANES
    n_ranges = _NSUB // heads
    rng_nodes = np_ // n_ranges
    eb_len = ebounds.shape[0]

    mesh = plsc.VectorSubcoreMesh(core_axis_name="c", subcore_axis_name="s")

    @functools.partial(
        pl.kernel,
        mesh=mesh,
        compiler_params=pltpu.CompilerParams(needs_layout_passes=False),
        out_type=jax.ShapeDtypeStruct((np_, hc), jnp.float32),
        scratch_types=[
            pltpu.VMEM((eb_len,), jnp.int32),
            pltpu.VMEM((hc,), jnp.float32),
            pltpu.VMEM((hc,), jnp.float32),
            pltpu.VMEM((_K,), jnp.int32),
            pltpu.VMEM((_K,), jnp.int32),
            pltpu.VMEM((_K + _LANES,), jnp.int32),
            pltpu.VMEM((_K + _LANES,), jnp.int32),
            pltpu.VMEM((_K,), jnp.int32),
            pltpu.VMEM((_K,), jnp.int32),
            pltpu.VMEM((_K,), jnp.int32),
            pltpu.VMEM((_K,), jnp.int32),
            pltpu.VMEM((_K, ch), jnp.float32),
            pltpu.VMEM((_K, ch), jnp.float32),
            pltpu.VMEM((_K, ch), jnp.float32),
            pltpu.VMEM((_K, ch), jnp.float32),
            pltpu.VMEM((ch,), jnp.float32),
            pltpu.VMEM((2 * _LANES,), jnp.float32),
            pltpu.VMEM((2, ch), jnp.float32),
            pltpu.VMEM((8, ch), jnp.float32),
            pltpu.SemaphoreType.DMA,
            pltpu.SemaphoreType.DMA,
            pltpu.SemaphoreType.DMA,
            pltpu.SemaphoreType.DMA,
            pltpu.SemaphoreType.DMA,
        ],
    )
    def edge_kernel(xl_h, xr_h, src_h, dst_h, eb_h, att_h, b_h, out_h,
                    eb_v, att_v, b_v, sidx0, sidx1, dstb0, dstb1,
                    gl0, gl1, gr0, gr1, rl0, rl1, rr0, rr1,
                    acc_v, st_v, orow_v, pre_v,
                    sl0, sl1, sr0, sr1, sem_out):
        sidx = (sidx0, sidx1)
        dstb = (dstb0, dstb1)
        gl = (gl0, gl1)
        gr = (gr0, gr1)
        rl = (rl0, rl1)
        rr = (rr0, rr1)
        sml = (sl0, sl1)
        smr = (sr0, sr1)

        wid = lax.axis_index("s") * 2 + lax.axis_index("c")
        rid = wid // heads
        h = wid - rid * heads
        hch = h * ch
        node0 = rid * rng_nodes
        pltpu.sync_copy(eb_h, eb_v)
        pltpu.sync_copy(att_h, att_v)
        pltpu.sync_copy(b_h, b_v)
        ebp = eb_v[pl.ds(rid * 8, _LANES)]
        e_lo = ebp[0]
        e_hi = ebp[1]
        hoffv = jnp.full((_LANES,), h * np_, jnp.int32)

        zero16 = jnp.zeros((_LANES,), jnp.float32)
        neg16 = jnp.full((_LANES,), -3e38, jnp.float32)
        for j in range(nj):
            acc_v[pl.ds(j * _LANES, _LANES)] = zero16

        for j in range(nj):
            bv = b_v[pl.ds(hch + j * _LANES, _LANES)]
            for r_ in range(8):
                pre_v[r_, pl.ds(j * _LANES, _LANES)] = bv

        def _pre_blk(t, _):
            pltpu.sync_copy(pre_v,
                            out_h.at[pl.ds(node0 + t * 8, 8), pl.ds(hch, ch)])
            return 0
        lax.fori_loop(0, rng_nodes // 8, _pre_blk, 0)

        lane15 = jnp.full((_LANES, 1), 15, jnp.int32)
        _gd = lax.GatherDimensionNumbers(
            offset_dims=(), collapsed_slice_dims=(0,), start_index_map=(0,))

        def _bcast_last(vec):
            return lax.gather(vec, lane15, _gd, slice_sizes=(1,),
                              mode=lax.GatherScatterMode.PROMISE_IN_BOUNDS)

        g0 = e_lo // _K
        g1 = (e_hi + (_K - 1)) // _K

        def _issue(g, b):
            base_e = g * _K
            pltpu.sync_copy(src_h.at[pl.ds(base_e, _K)], sidx[b])
            pltpu.sync_copy(dst_h.at[pl.ds(base_e, _K)],
                            dstb[b].at[pl.ds(0, _K)])
            for q in range(_K // _LANES):
                sl_ = pl.ds(q * _LANES, _LANES)
                gl[b][sl_] = sidx[b][sl_] + hoffv
                gr[b][sl_] = dstb[b][sl_] + hoffv
            pltpu.async_copy(xl_h.at[gl[b]], rl[b], sml[b])
            pltpu.async_copy(xr_h.at[gr[b]], rr[b], smr[b])

        def _chunk(g, b, cf):
            @pl.when(g + 1 < g1)
            def _():
                _issue(g + 1, 1 - b)

            pltpu.make_async_copy(xl_h.at[gl[b]], rl[b], sml[b]).wait()
            pltpu.make_async_copy(xr_h.at[gr[b]], rr[b], smr[b]).wait()
            base_e = g * _K
            dv = dstb[b]
            rlv = rl[b]
            rrv = rr[b]
            att_vs = [att_v[pl.ds(hch + j * _LANES, _LANES)]
                      for j in range(nj)]

            def edge_body(i, carry):
                cur = carry[0]
                fcnt = carry[1]
                m = carry[2]
                den = carry[3]
                accs = carry[4:]
                d = dv[pl.ds(i, _LANES)][0]
                is_new = d != cur

                def flush(cf0):
                    cur0, f0 = cf0

                    @pl.when(cur0 >= 0)
                    def _():
                        slot = lax.rem(f0, 2)

                        @pl.when(f0 >= 2)
                        def _():
                            pltpu.make_async_copy(
                                orow_v.at[slot],
                                out_h.at[cur0, pl.ds(hch, ch)],
                                sem_out).wait()

                        inv = 1.0 / (den + 1e-16)
                        for j in range(nj):
                            orow_v[slot, pl.ds(j * _LANES, _LANES)] = (
                                accs[j] * inv
                                + b_v[pl.ds(hch + j * _LANES, _LANES)])
                        pltpu.async_copy(orow_v.at[slot],
                                         out_h.at[cur0, pl.ds(hch, ch)],
                                         sem_out)
                    return (d, jnp.where(cur0 >= 0, f0 + 1, f0))

                cur, fcnt = lax.cond(is_new, flush, lambda c2: c2,
                                     (cur, fcnt))

                m_b = jnp.where(is_new, neg16, m)
                den_b = jnp.where(is_new, zero16, den)
                parts = [zero16] * 8
                ls = []
                for j in range(nj):
                    sl_ = pl.ds(j * _LANES, _LANES)
                    lj = rlv[i, sl_]
                    ls.append(lj)
                    z = lj + rrv[i, sl_]
                    z = jnp.maximum(z, 0.2 * z)
                    parts[j % 8] = parts[j % 8] + att_vs[j] * z
                part = parts[0]
                if nj > 8:
                    part = (((parts[0] + parts[1]) + (parts[2] + parts[3]))
                            + ((parts[4] + parts[5]) + (parts[6] + parts[7])))
                lvec = _bcast_last(jnp.cumsum(part))
                mn = jnp.maximum(m_b, lvec)
                r = jnp.exp(m_b - mn)
                w = jnp.exp(lvec - mn)
                den_n = den_b * r + w
                r_eff = jnp.where(is_new, zero16, r)
                new_accs = [accs[j] * r_eff + w * ls[j] for j in range(nj)]
                return (cur, fcnt, mn, den_n, *new_accs)

            ilo = jnp.maximum(e_lo - base_e, 0)
            ihi = jnp.minimum(e_hi - base_e, _K)
            m0 = st_v[pl.ds(0, _LANES)]
            d0 = st_v[pl.ds(_LANES, _LANES)]
            accs0 = [acc_v[pl.ds(j * _LANES, _LANES)] for j in range(nj)]
            res = lax.fori_loop(ilo, ihi, edge_body,
                                (cf[0], cf[1], m0, d0, *accs0))
            st_v[pl.ds(0, _LANES)] = res[2]
            st_v[pl.ds(_LANES, _LANES)] = res[3]
            for j in range(nj):
                acc_v[pl.ds(j * _LANES, _LANES)] = res[4 + j]
            return (res[0], res[1])

        @pl.when(g1 > g0)
        def _():
            _issue(g0, 0)

        def pair_body(t, cf):
            for b in (0, 1):
                g = g0 + 2 * t + b
                cf = lax.cond(g < g1,
                              lambda c, g=g, b=b: _chunk(g, b, c),
                              lambda c: c, cf)
            return cf

        npairs = (g1 - g0 + 1) // 2
        cur, fcnt = lax.fori_loop(0, npairs, pair_body,
                                  (jnp.int32(-1), jnp.int32(0)))

        @pl.when(cur >= 0)
        def _():
            slot = lax.rem(fcnt, 2)

            @pl.when(fcnt >= 2)
            def _():
                pltpu.make_async_copy(orow_v.at[slot],
                                      out_h.at[cur, pl.ds(hch, ch)],
                                      sem_out).wait()

            den = st_v[pl.ds(_LANES, _LANES)]
            inv = 1.0 / (den + 1e-16)
            for j in range(nj):
                orow_v[slot, pl.ds(j * _LANES, _LANES)] = (
                    acc_v[pl.ds(j * _LANES, _LANES)] * inv
                    + b_v[pl.ds(hch + j * _LANES, _LANES)])
            pltpu.async_copy(orow_v.at[slot],
                             out_h.at[cur, pl.ds(hch, ch)], sem_out)

        fin = fcnt + jnp.where(cur >= 0, 1, 0)

        @pl.when(fin >= 1)
        def _():
            pltpu.make_async_copy(orow_v.at[0],
                                  out_h.at[0, pl.ds(hch, ch)],
                                  sem_out).wait()

        @pl.when(fin >= 2)
        def _():
            pltpu.make_async_copy(orow_v.at[0],
                                  out_h.at[0, pl.ds(hch, ch)],
                                  sem_out).wait()

    return edge_kernel(xl2, xr2, src_s, dst_s, ebounds, att_f, bias)


def _head_major(x, heads, ch, np_):
    if heads == 1:
        return x
    return x.reshape(np_, heads, ch).transpose(1, 0, 2).reshape(
        heads * np_, ch)


def kernel(x, edge_index, W1l, W1r, att1, b1, W2l, W2r, att2, b2,
           W3l, W3r, att3, b3):
    n = x.shape[0]
    e = edge_index.shape[1]

    npw = ((n + _NSUB - 1) // _NSUB + 7) // 8 * 8
    np_ = ((npw * _NSUB + 127) // 128) * 128
    npw = np_ // _NSUB

    src = edge_index[0].astype(jnp.int32)
    dst = edge_index[1].astype(jnp.int32)
    order = jnp.argsort(dst)
    src_s = jnp.take(src, order)
    dst_s = jnp.take(dst, order)
    ep = (e + _K - 1) // _K * _K
    if ep != e:
        src_s = jnp.pad(src_s, (0, ep - e))
        dst_s = jnp.pad(dst_s, (0, ep - e), constant_values=n)
    bounds = jnp.arange(_NSUB + 1, dtype=jnp.int32) * npw
    estarts = jnp.searchsorted(dst_s[:e], bounds, side="left").astype(jnp.int32)

    def _ebounds(heads):
        nr = _NSUB // heads
        idx = jnp.arange(nr, dtype=jnp.int32)
        eb = jnp.zeros((nr * 8 + 16,), jnp.int32)
        eb = eb.at[idx * 8].set(estarts[idx * heads])
        eb = eb.at[idx * 8 + 1].set(estarts[(idx + 1) * heads])
        return eb

    xp = jnp.pad(x, ((0, np_ - n), (0, 0)))

    w1 = jnp.concatenate([W1l, W1r], axis=1)
    w2 = jnp.concatenate([W2l, W2r], axis=1)
    w3 = jnp.concatenate([W3l, W3r], axis=1)

    outs = []
    a = xp
    for li, (w, att, b) in enumerate(
            ((w1, att1, b1), (w2, att2, b2), (w3, att3, b3))):
        heads, ch = att.shape
        xl, xr = _matmul(a, w, hc=heads * ch, elu=(li > 0))
        xl2 = _head_major(xl, heads, ch, np_)
        xr2 = _head_major(xr, heads, ch, np_)
        a = _edge_stage(xl2, xr2, src_s, dst_s, _ebounds(heads),
                        att.reshape(-1), b, heads, ch, np_)
    return a[:n]


# speedup vs baseline: 1.2021x; 1.0406x over previous
import functools

import jax
import jax.numpy as jnp
from jax import lax
from jax.experimental import pallas as pl
from jax.experimental.pallas import tpu as pltpu
from jax.experimental.pallas import tpu_sc as plsc

_LANES = 16
_NSUB = 32
_K = 96


def _mm_body(a_ref, w_ref, xl_ref, xr_ref, *, hc, elu):
    a = a_ref[...]
    if elu:
        a = jnp.where(a > 0.0, a, jnp.exp(jnp.minimum(a, 0.0)) - 1.0)
    o = lax.dot(a, w_ref[...], preferred_element_type=jnp.float32)
    xl_ref[...] = o[:, :hc]
    xr_ref[...] = o[:, hc:]


def _matmul(a, w, hc, elu):
    np_, kdim = a.shape
    bm = 128
    grid = np_ // bm
    return pl.pallas_call(
        functools.partial(_mm_body, hc=hc, elu=elu),
        grid=(grid,),
        in_specs=[
            pl.BlockSpec((bm, kdim), lambda i: (i, 0)),
            pl.BlockSpec((kdim, 2 * hc), lambda i: (0, 0)),
        ],
        out_specs=[
            pl.BlockSpec((bm, hc), lambda i: (i, 0)),
            pl.BlockSpec((bm, hc), lambda i: (i, 0)),
        ],
        out_shape=[
            jax.ShapeDtypeStruct((np_, hc), jnp.float32),
            jax.ShapeDtypeStruct((np_, hc), jnp.float32),
        ],
    )(a, w)


def _edge_stage(xl2, xr2, src_s, dst_s, ebounds, att_f, bias, heads, ch, np_):
    hc = heads * ch
    nj = ch // _LANES
    n_ranges = _NSUB // heads
    rng_nodes = np_ // n_ranges
    eb_len = ebounds.shape[0]

    mesh = plsc.VectorSubcoreMesh(core_axis_name="c", subcore_axis_name="s")

    @functools.partial(
        pl.kernel,
        mesh=mesh,
        compiler_params=pltpu.CompilerParams(needs_layout_passes=False),
        out_type=jax.ShapeDtypeStruct((np_, hc), jnp.float32),
        scratch_types=[
            pltpu.VMEM((eb_len,), jnp.int32),
            pltpu.VMEM((hc,), jnp.float32),
            pltpu.VMEM((hc,), jnp.float32),
            pltpu.VMEM((_K,), jnp.int32),
            pltpu.VMEM((_K,), jnp.int32),
            pltpu.VMEM((_K + _LANES,), jnp.int32),
            pltpu.VMEM((_K + _LANES,), jnp.int32),
            pltpu.VMEM((_K,), jnp.int32),
            pltpu.VMEM((_K,), jnp.int32),
            pltpu.VMEM((_K,), jnp.int32),
            pltpu.VMEM((_K,), jnp.int32),
            pltpu.VMEM((_K, ch), jnp.float32),
            pltpu.VMEM((_K, ch), jnp.float32),
            pltpu.VMEM((_K, ch), jnp.float32),
            pltpu.VMEM((_K, ch), jnp.float32),
            pltpu.VMEM((ch,), jnp.float32),
            pltpu.VMEM((2 * _LANES,), jnp.float32),
            pltpu.VMEM((2, ch), jnp.float32),
            pltpu.VMEM((8, ch), jnp.float32),
            pltpu.SemaphoreType.DMA,
            pltpu.SemaphoreType.DMA,
            pltpu.SemaphoreType.DMA,
            pltpu.SemaphoreType.DMA,
            pltpu.SemaphoreType.DMA,
        ],
    )
    def edge_kernel(xl_h, xr_h, src_h, dst_h, eb_h, att_h, b_h, out_h,
                    eb_v, att_v, b_v, sidx0, sidx1, dstb0, dstb1,
                    gl0, gl1, gr0, gr1, rl0, rl1, rr0, rr1,
                    acc_v, st_v, orow_v, pre_v,
                    sl0, sl1, sr0, sr1, sem_out):
        sidx = (sidx0, sidx1)
        dstb = (dstb0, dstb1)
        gl = (gl0, gl1)
        gr = (gr0, gr1)
        rl = (rl0, rl1)
        rr = (rr0, rr1)
        sml = (sl0, sl1)
        smr = (sr0, sr1)

        wid = lax.axis_index("s") * 2 + lax.axis_index("c")
        rid = wid // heads
        h = wid - rid * heads
        hch = h * ch
        node0 = rid * rng_nodes
        pltpu.sync_copy(eb_h, eb_v)
        pltpu.sync_copy(att_h, att_v)
        pltpu.sync_copy(b_h, b_v)
        ebp = eb_v[pl.ds(rid * 8, _LANES)]
        e_lo = ebp[0]
        e_hi = ebp[1]
        hoffv = jnp.full((_LANES,), h * np_, jnp.int32)

        zero16 = jnp.zeros((_LANES,), jnp.float32)
        neg16 = jnp.full((_LANES,), -3e38, jnp.float32)
        for j in range(nj):
            acc_v[pl.ds(j * _LANES, _LANES)] = zero16

        for j in range(nj):
            bv = b_v[pl.ds(hch + j * _LANES, _LANES)]
            for r_ in range(8):
                pre_v[r_, pl.ds(j * _LANES, _LANES)] = bv

        def _pre_blk(t, _):
            pltpu.sync_copy(pre_v,
                            out_h.at[pl.ds(node0 + t * 8, 8), pl.ds(hch, ch)])
            return 0
        lax.fori_loop(0, rng_nodes // 8, _pre_blk, 0)

        lane15 = jnp.full((_LANES, 1), 15, jnp.int32)
        _gd = lax.GatherDimensionNumbers(
            offset_dims=(), collapsed_slice_dims=(0,), start_index_map=(0,))

        def _bcast_last(vec):
            return lax.gather(vec, lane15, _gd, slice_sizes=(1,),
                              mode=lax.GatherScatterMode.PROMISE_IN_BOUNDS)

        g0 = e_lo // _K
        g1 = (e_hi + (_K - 1)) // _K

        def _issue(g, b):
            base_e = g * _K
            pltpu.sync_copy(src_h.at[pl.ds(base_e, _K)], sidx[b])
            pltpu.sync_copy(dst_h.at[pl.ds(base_e, _K)],
                            dstb[b].at[pl.ds(0, _K)])
            for q in range(_K // _LANES):
                sl_ = pl.ds(q * _LANES, _LANES)
                gl[b][sl_] = sidx[b][sl_] + hoffv
                gr[b][sl_] = dstb[b][sl_] + hoffv
            pltpu.async_copy(xl_h.at[gl[b]], rl[b], sml[b])
            pltpu.async_copy(xr_h.at[gr[b]], rr[b], smr[b])

        def _chunk(g, b, cf):
            @pl.when(g + 1 < g1)
            def _():
                _issue(g + 1, 1 - b)

            pltpu.make_async_copy(xl_h.at[gl[b]], rl[b], sml[b]).wait()
            pltpu.make_async_copy(xr_h.at[gr[b]], rr[b], smr[b]).wait()
            base_e = g * _K
            dv = dstb[b]
            rlv = rl[b]
            rrv = rr[b]
            att_vs = [att_v[pl.ds(hch + j * _LANES, _LANES)]
                      for j in range(nj)]

            def edge_body(i, carry):
                cur = carry[0]
                fcnt = carry[1]
                m = carry[2]
                den = carry[3]
                accs = carry[4:]
                d = dv[pl.ds(i, _LANES)][0]
                is_new = d != cur

                def flush(cf0):
                    cur0, f0 = cf0

                    @pl.when(cur0 >= 0)
                    def _():
                        slot = lax.rem(f0, 2)

                        @pl.when(f0 >= 2)
                        def _():
                            pltpu.make_async_copy(
                                orow_v.at[slot],
                                out_h.at[cur0, pl.ds(hch, ch)],
                                sem_out).wait()

                        inv = 1.0 / (den + 1e-16)
                        for j in range(nj):
                            orow_v[slot, pl.ds(j * _LANES, _LANES)] = (
                                accs[j] * inv
                                + b_v[pl.ds(hch + j * _LANES, _LANES)])
                        pltpu.async_copy(orow_v.at[slot],
                                         out_h.at[cur0, pl.ds(hch, ch)],
                                         sem_out)
                    return (d, jnp.where(cur0 >= 0, f0 + 1, f0))

                cur, fcnt = lax.cond(is_new, flush, lambda c2: c2,
                                     (cur, fcnt))

                m_b = jnp.where(is_new, neg16, m)
                den_b = jnp.where(is_new, zero16, den)
                parts = [zero16] * 8
                ls = []
                for j in range(nj):
                    sl_ = pl.ds(j * _LANES, _LANES)
                    lj = rlv[i, sl_]
                    ls.append(lj)
                    z = lj + rrv[i, sl_]
                    z = jnp.maximum(z, 0.2 * z)
                    parts[j % 8] = parts[j % 8] + att_vs[j] * z
                part = parts[0]
                if nj > 8:
                    part = (((parts[0] + parts[1]) + (parts[2] + parts[3]))
                            + ((parts[4] + parts[5]) + (parts[6] + parts[7])))
                lvec = _bcast_last(jnp.cumsum(part))
                mn = jnp.maximum(m_b, lvec)
                r = jnp.exp(m_b - mn)
                w = jnp.exp(lvec - mn)
                den_n = den_b * r + w
                r_eff = jnp.where(is_new, zero16, r)
                new_accs = [accs[j] * r_eff + w * ls[j] for j in range(nj)]
                return (cur, fcnt, mn, den_n, *new_accs)

            ilo = jnp.maximum(e_lo - base_e, 0)
            ihi = jnp.minimum(e_hi - base_e, _K)
            m0 = st_v[pl.ds(0, _LANES)]
            d0 = st_v[pl.ds(_LANES, _LANES)]
            accs0 = [acc_v[pl.ds(j * _LANES, _LANES)] for j in range(nj)]
            res = lax.fori_loop(ilo, ihi, edge_body,
                                (cf[0], cf[1], m0, d0, *accs0))
            st_v[pl.ds(0, _LANES)] = res[2]
            st_v[pl.ds(_LANES, _LANES)] = res[3]
            for j in range(nj):
                acc_v[pl.ds(j * _LANES, _LANES)] = res[4 + j]
            return (res[0], res[1])

        @pl.when(g1 > g0)
        def _():
            _issue(g0, 0)

        def pair_body(t, cf):
            for b in (0, 1):
                g = g0 + 2 * t + b
                cf = lax.cond(g < g1,
                              lambda c, g=g, b=b: _chunk(g, b, c),
                              lambda c: c, cf)
            return cf

        npairs = (g1 - g0 + 1) // 2
        cur, fcnt = lax.fori_loop(0, npairs, pair_body,
                                  (jnp.int32(-1), jnp.int32(0)))

        @pl.when(cur >= 0)
        def _():
            slot = lax.rem(fcnt, 2)

            @pl.when(fcnt >= 2)
            def _():
                pltpu.make_async_copy(orow_v.at[slot],
                                      out_h.at[cur, pl.ds(hch, ch)],
                                      sem_out).wait()

            den = st_v[pl.ds(_LANES, _LANES)]
            inv = 1.0 / (den + 1e-16)
            for j in range(nj):
                orow_v[slot, pl.ds(j * _LANES, _LANES)] = (
                    acc_v[pl.ds(j * _LANES, _LANES)] * inv
                    + b_v[pl.ds(hch + j * _LANES, _LANES)])
            pltpu.async_copy(orow_v.at[slot],
                             out_h.at[cur, pl.ds(hch, ch)], sem_out)

        fin = fcnt + jnp.where(cur >= 0, 1, 0)

        @pl.when(fin >= 1)
        def _():
            pltpu.make_async_copy(orow_v.at[0],
                                  out_h.at[0, pl.ds(hch, ch)],
                                  sem_out).wait()

        @pl.when(fin >= 2)
        def _():
            pltpu.make_async_copy(orow_v.at[0],
                                  out_h.at[0, pl.ds(hch, ch)],
                                  sem_out).wait()

    return edge_kernel(xl2, xr2, src_s, dst_s, ebounds, att_f, bias)


def _head_major(x, heads, ch, np_):
    if heads == 1:
        return x
    return x.reshape(np_, heads, ch).transpose(1, 0, 2).reshape(
        heads * np_, ch)


def kernel(x, edge_index, W1l, W1r, att1, b1, W2l, W2r, att2, b2,
           W3l, W3r, att3, b3):
    n = x.shape[0]
    e = edge_index.shape[1]

    npw = ((n + _NSUB - 1) // _NSUB + 7) // 8 * 8
    np_ = ((npw * _NSUB + 127) // 128) * 128
    npw = np_ // _NSUB

    src = edge_index[0].astype(jnp.int32)
    dst = edge_index[1].astype(jnp.int32)
    order = jnp.argsort(dst)
    src_s = jnp.take(src, order)
    dst_s = jnp.take(dst, order)
    ep = (e + _K - 1) // _K * _K
    if ep != e:
        src_s = jnp.pad(src_s, (0, ep - e))
        dst_s = jnp.pad(dst_s, (0, ep - e), constant_values=n)
    bounds = jnp.arange(_NSUB + 1, dtype=jnp.int32) * npw
    estarts = jnp.searchsorted(dst_s[:e], bounds, side="left").astype(jnp.int32)

    def _ebounds(heads):
        nr = _NSUB // heads
        idx = jnp.arange(nr, dtype=jnp.int32)
        eb = jnp.zeros((nr * 8 + 16,), jnp.int32)
        eb = eb.at[idx * 8].set(estarts[idx * heads])
        eb = eb.at[idx * 8 + 1].set(estarts[(idx + 1) * heads])
        return eb

    xp = jnp.pad(x, ((0, np_ - n), (0, 0)))

    w1 = jnp.concatenate([W1l, W1r], axis=1)
    w2 = jnp.concatenate([W2l, W2r], axis=1)
    w3 = jnp.concatenate([W3l, W3r], axis=1)

    outs = []
    a = xp
    for li, (w, att, b) in enumerate(
            ((w1, att1, b1), (w2, att2, b2), (w3, att3, b3))):
        heads, ch = att.shape
        xl, xr = _matmul(a, w, hc=heads * ch, elu=(li > 0))
        xl2 = _head_major(xl, heads, ch, np_)
        xr2 = _head_major(xr, heads, ch, np_)
        a = _edge_stage(xl2, xr2, src_s, dst_s, _ebounds(heads),
                        att.reshape(-1), b, heads, ch, np_)
    return a[:n]
